# Initial kernel scaffold; baseline (speedup 1.0000x reference)
import jax
import jax.numpy as jnp
from jax.experimental import pallas as pl


def kernel(position_ids, table):
    raise NotImplementedError("write your pallas kernel here")


# speedup vs baseline: 2.1928x; 2.1928x over previous
import functools

import jax
import jax.numpy as jnp
from jax import lax
from jax.experimental import pallas as pl
from jax.experimental.pallas import tpu as pltpu
from jax.experimental.pallas import tpu_sc as plsc

D = 1024
NC, NS = 2, 16
NW = NC * NS
CHUNK = 64


def _make_gather(n_idx):
    b_per_w = n_idx // NW
    n_chunks = b_per_w // CHUNK
    mesh = plsc.VectorSubcoreMesh(core_axis_name="c", subcore_axis_name="s")

    @functools.partial(
        pl.kernel,
        mesh=mesh,
        out_type=jax.ShapeDtypeStruct((n_idx, D), jnp.float32),
        scratch_types=[
            pltpu.VMEM((b_per_w,), jnp.int32),
            pltpu.VMEM((CHUNK, D), jnp.float32),
            pltpu.SemaphoreType.DMA,
        ],
    )
    def gather(pos_hbm, table_hbm, out_hbm, idx_v, rows_v, sem):
        wid = lax.axis_index("s") * NC + lax.axis_index("c")
        base = wid * b_per_w
        pltpu.sync_copy(pos_hbm.at[pl.ds(base, b_per_w)], idx_v)

        def step(i, _):
            off = i * CHUNK
            pltpu.async_copy(
                table_hbm.at[idx_v.at[pl.ds(off, CHUNK)]], rows_v, sem
            ).wait()
            pltpu.sync_copy(rows_v, out_hbm.at[pl.ds(base + off, CHUNK)])
            return 0

        lax.fori_loop(0, n_chunks, step, 0)

    return gather


def kernel(position_ids, table):
    pos = position_ids.reshape(-1)
    out = _make_gather(pos.shape[0])(pos, table)
    return out.reshape(position_ids.shape + (table.shape[1],))


# speedup vs baseline: 2.2498x; 1.0260x over previous
import functools

import jax
import jax.numpy as jnp
from jax import lax
from jax.experimental import pallas as pl
from jax.experimental.pallas import tpu as pltpu
from jax.experimental.pallas import tpu_sc as plsc

D = 1024
NC, NS = 2, 16
NW = NC * NS
CHUNK = 32


def _make_gather(n_idx):
    b_per_w = n_idx // NW
    n_pairs = b_per_w // (2 * CHUNK)
    mesh = plsc.VectorSubcoreMesh(core_axis_name="c", subcore_axis_name="s")

    @functools.partial(
        pl.kernel,
        mesh=mesh,
        out_type=jax.ShapeDtypeStruct((n_idx, D), jnp.float32),
        scratch_types=[
            pltpu.VMEM((b_per_w,), jnp.int32),
            pltpu.VMEM((CHUNK, D), jnp.float32),
            pltpu.VMEM((CHUNK, D), jnp.float32),
            pltpu.SemaphoreType.DMA,
            pltpu.SemaphoreType.DMA,
            pltpu.SemaphoreType.DMA,
            pltpu.SemaphoreType.DMA,
        ],
    )
    def gather(pos_hbm, table_hbm, out_hbm, idx_v, r0, r1, gs0, gs1, ws0, ws1):
        wid = lax.axis_index("s") * NC + lax.axis_index("c")
        base = wid * b_per_w
        pltpu.sync_copy(pos_hbm.at[pl.ds(base, b_per_w)], idx_v)

        def g_src(i):
            return table_hbm.at[idx_v.at[pl.ds(i * CHUNK, CHUNK)]]

        def w_dst(i):
            return out_hbm.at[pl.ds(base + i * CHUNK, CHUNK)]

        pltpu.async_copy(g_src(0), r0, gs0)
        pltpu.async_copy(g_src(1), r1, gs1)

        def step(p, _):
            i = 2 * p
            pltpu.make_async_copy(g_src(i), r0, gs0).wait()
            pltpu.async_copy(r0, w_dst(i), ws0)
            pltpu.make_async_copy(g_src(i + 1), r1, gs1).wait()
            pltpu.async_copy(r1, w_dst(i + 1), ws1)

            @pl.when(p + 1 < n_pairs)
            def _():
                pltpu.make_async_copy(r0, w_dst(i), ws0).wait()
                pltpu.async_copy(g_src(i + 2), r0, gs0)
                pltpu.make_async_copy(r1, w_dst(i + 1), ws1).wait()
                pltpu.async_copy(g_src(i + 3), r1, gs1)

            return 0

        lax.fori_loop(0, n_pairs, step, 0)

        last = 2 * (n_pairs - 1)
        pltpu.make_async_copy(r0, w_dst(last), ws0).wait()
        pltpu.make_async_copy(r1, w_dst(last + 1), ws1).wait()

    return gather


def kernel(position_ids, table):
    pos = position_ids.reshape(-1)
    out = _make_gather(pos.shape[0])(pos, table)
    return out.reshape(position_ids.shape + (table.shape[1],))
